# baseline (speedup 1.0000x reference)
import functools
import math

import jax
import jax.numpy as jnp
import numpy as np
from jax import lax
from jax.experimental import pallas as pl
from jax.experimental.pallas import tpu as pltpu
from jax.experimental.pallas import tpu_sc as plsc

VOCAB = 100000
D_MODEL = 512
MAX_LEN = 2048
BATCH = 4
SEQ = 2048

NUM_CORES = 2
NUM_SUBCORES = 16
NW = NUM_CORES * NUM_SUBCORES
ROWS = BATCH * SEQ
RPW = ROWS // NW
CHUNK = 64
LANES = 16
SCALE = math.sqrt(D_MODEL)


def _make_pe():
    position = np.arange(MAX_LEN, dtype=np.float32)[:, None]
    div_term = np.exp(
        np.arange(0, D_MODEL, 2, dtype=np.float32) * (-math.log(10000.0) / D_MODEL)
    )
    pe = np.zeros((MAX_LEN, D_MODEL), dtype=np.float32)
    pe[:, 0::2] = np.sin(position * div_term)
    pe[:, 1::2] = np.cos(position * div_term)
    return pe


_PE = _make_pe()

_mesh = plsc.VectorSubcoreMesh(core_axis_name="c", subcore_axis_name="s")


@functools.partial(
    pl.kernel,
    mesh=_mesh,
    out_type=jax.ShapeDtypeStruct((ROWS, D_MODEL), jnp.float32),
    scratch_types=[
        pltpu.VMEM((RPW,), jnp.int32),
        pltpu.VMEM((CHUNK, D_MODEL), jnp.float32),
        pltpu.VMEM((CHUNK, D_MODEL), jnp.float32),
        pltpu.SemaphoreType.DMA,
    ],
)
def _embed(x_hbm, table_hbm, pe_hbm, out_hbm, idx_v, rows_v, pe_v, sem):
    wid = lax.axis_index("s") * NUM_CORES + lax.axis_index("c")
    base = wid * RPW
    s0 = lax.rem(base, SEQ)

    pltpu.sync_copy(x_hbm.at[pl.ds(base, RPW)], idx_v)

    def chunk_body(ci, carry):
        r0 = ci * CHUNK
        gather = pltpu.async_copy(
            table_hbm.at[idx_v.at[pl.ds(r0, CHUNK)]], rows_v, sem
        )
        pltpu.sync_copy(pe_hbm.at[pl.ds(s0 + r0, CHUNK)], pe_v)
        gather.wait()

        def row_body(r, c2):
            for j in range(D_MODEL // LANES):
                sl = pl.ds(j * LANES, LANES)
                rows_v[r, sl] = rows_v[r, sl] * SCALE + pe_v[r, sl]
            return c2

        lax.fori_loop(0, CHUNK, row_body, 0)
        pltpu.sync_copy(rows_v, out_hbm.at[pl.ds(base + r0, CHUNK)])
        return carry

    lax.fori_loop(0, RPW // CHUNK, chunk_body, 0)


def kernel(x, table):
    out = _embed(x.reshape(ROWS), table, jnp.asarray(_PE))
    return out.reshape(BATCH, SEQ, D_MODEL)


# speedup vs baseline: 1.2735x; 1.2735x over previous
import functools
import math

import jax
import jax.numpy as jnp
import numpy as np
from jax import lax
from jax.experimental import pallas as pl
from jax.experimental.pallas import tpu as pltpu
from jax.experimental.pallas import tpu_sc as plsc

VOCAB = 100000
D_MODEL = 512
MAX_LEN = 2048
BATCH = 4
SEQ = 2048

NUM_CORES = 2
NUM_SUBCORES = 16
NW = NUM_CORES * NUM_SUBCORES
PPW = SEQ // NW
LANES = 16
VPR = D_MODEL // LANES
SCALE = math.sqrt(D_MODEL)


def _make_pe():
    position = np.arange(MAX_LEN, dtype=np.float32)[:, None]
    div_term = np.exp(
        np.arange(0, D_MODEL, 2, dtype=np.float32) * (-math.log(10000.0) / D_MODEL)
    )
    pe = np.zeros((MAX_LEN, D_MODEL), dtype=np.float32)
    pe[:, 0::2] = np.sin(position * div_term)
    pe[:, 1::2] = np.cos(position * div_term)
    return pe


_PE = _make_pe()

_mesh = plsc.VectorSubcoreMesh(core_axis_name="c", subcore_axis_name="s")


@functools.partial(
    pl.kernel,
    mesh=_mesh,
    out_type=jax.ShapeDtypeStruct((BATCH * SEQ, D_MODEL), jnp.float32),
    scratch_types=[
        pltpu.VMEM((BATCH, PPW), jnp.int32),
        pltpu.VMEM((PPW, D_MODEL), jnp.float32),
        pltpu.VMEM((PPW, D_MODEL), jnp.float32),
        pltpu.VMEM((PPW, D_MODEL), jnp.float32),
        pltpu.SemaphoreType.DMA,
        pltpu.SemaphoreType.DMA,
        pltpu.SemaphoreType.DMA,
        pltpu.SemaphoreType.DMA,
    ],
)
def _embed(x_hbm, table_hbm, pe_hbm, out_hbm,
           idx_v, pe_v, buf0, buf1, sg0, sg1, so0, so1):
    wid = lax.axis_index("s") * NUM_CORES + lax.axis_index("c")
    p0 = wid * PPW

    bufs = (buf0, buf1)
    gsems = (sg0, sg1)
    osems = (so0, so1)

    for b in range(BATCH):
        pltpu.sync_copy(x_hbm.at[pl.ds(b * SEQ + p0, PPW)], idx_v.at[b])

    gathers = [None] * BATCH
    outs = [None] * BATCH
    gathers[0] = pltpu.async_copy(table_hbm.at[idx_v.at[0]], bufs[0], gsems[0])
    pltpu.sync_copy(pe_hbm.at[pl.ds(p0, PPW)], pe_v)

    for b in range(BATCH):
        p = b % 2
        q = (b + 1) % 2
        if b + 1 < BATCH:
            if b >= 1:
                outs[b - 1].wait()
            gathers[b + 1] = pltpu.async_copy(
                table_hbm.at[idx_v.at[b + 1]], bufs[q], gsems[q]
            )
        gathers[b].wait()

        def row_body(r, carry, _buf=bufs[p]):
            for j in range(VPR):
                sl = pl.ds(j * LANES, LANES)
                _buf[r, sl] = _buf[r, sl] * SCALE + pe_v[r, sl]
            return carry

        lax.fori_loop(0, PPW, row_body, 0)
        outs[b] = pltpu.async_copy(
            bufs[p], out_hbm.at[pl.ds(b * SEQ + p0, PPW)], osems[p]
        )

    outs[BATCH - 2].wait()
    outs[BATCH - 1].wait()


def kernel(x, table):
    out = _embed(x.reshape(BATCH * SEQ), table, jnp.asarray(_PE))
    return out.reshape(BATCH, SEQ, D_MODEL)
